# baseline (speedup 1.0000x reference)
import functools

import jax
import jax.numpy as jnp
from jax.experimental import pallas as pl
from jax.experimental.pallas import tpu as pltpu

EPS = 1e-5
LANE = 128

_CP = pltpu.CompilerParams(
    dimension_semantics=("parallel",),
    vmem_limit_bytes=64 * 1024 * 1024,
)


def _rup(x, m):
    return (x + m - 1) // m * m


def _tile(m, target):
    if m <= target:
        return m
    for t in range(target, 7, -1):
        if m % t == 0 and t % 8 == 0:
            return t
    return m


def _pad_last(a, c):
    pad = c - a.shape[-1]
    if pad == 0:
        return a
    return jnp.pad(a, [(0, 0)] * (a.ndim - 1) + [(0, pad)])


def _stats(yf):
    return jnp.concatenate(
        [jnp.sum(yf, 0, keepdims=True), jnp.sum(yf * yf, 0, keepdims=True)], 0)


def _fold2(s, ss, count, gamma, beta):
    mean = s / count
    var = jnp.maximum(ss / count - mean * mean, 0.0)
    scale = gamma * jax.lax.rsqrt(var + EPS)
    return scale, beta - mean * scale


def _fold(st, count, gamma, beta):
    s = jnp.sum(st[:, 0, :], axis=0, keepdims=True)
    ss = jnp.sum(st[:, 1, :], axis=0, keepdims=True)
    return _fold2(s, ss, count, gamma, beta)


def _mm_stats_kernel(x_ref, w_ref, y_ref, st_ref):
    y = jnp.dot(x_ref[...], w_ref[...], preferred_element_type=jnp.float32)
    yb = y.astype(jnp.bfloat16)
    y_ref[...] = yb
    st_ref[0] = _stats(yb.astype(jnp.float32))


def _conv2_taps(z_ref, w2_ref, ho, wo):
    hw = ho * wo
    cp = z_ref.shape[-1]
    colmask = (jax.lax.broadcasted_iota(jnp.int32, (hw, 1), 0) % wo) != 0
    acc = jnp.zeros((hw, w2_ref.shape[-1]), jnp.float32)
    for ky in range(3):
        parts = []
        for kx in range(3):
            a, b = (ky + 1) % 2, (kx + 1) % 2
            base = z_ref[a * 2 + b]
            shift = (wo if ky == 0 else 0) + (1 if kx == 0 else 0)
            if shift:
                base = jnp.concatenate(
                    [jnp.zeros((shift, cp), base.dtype), base[:hw - shift]],
                    axis=0)
            if kx == 0:
                base = jnp.where(colmask, base, jnp.bfloat16(0))
            parts.append(base)
        wide = jnp.concatenate(parts, axis=1)
        acc = acc + jnp.dot(wide, w2_ref[ky],
                            preferred_element_type=jnp.float32)
    return acc


def _mega_kernel(y1_ref, st1_ref, xs_ref, w2_ref, w3_ref, ws_ref, gb_ref,
                 o_ref, z_ref, y2_ref, y3_ref, ysc_ref,
                 *, n, ho, wo, m1, m2, cp, cpo):
    hw = ho * wo
    s1, b1 = _fold(st1_ref[...], m1,
                   gb_ref[0:1, :cp], gb_ref[1:2, :cp])

    def body1(i, carry):
        s2s, ss2s = carry
        for p in range(4):
            z_ref[p] = jnp.maximum(
                y1_ref[i, p].astype(jnp.float32) * s1 + b1, 0.0
            ).astype(jnp.bfloat16)
        yb = _conv2_taps(z_ref, w2_ref, ho, wo).astype(jnp.bfloat16)
        y2_ref[i] = yb
        yf = yb.astype(jnp.float32)
        return (s2s + jnp.sum(yf, 0, keepdims=True),
                ss2s + jnp.sum(yf * yf, 0, keepdims=True))

    zero = jnp.zeros((1, cp), jnp.float32)
    s2s, ss2s = jax.lax.fori_loop(0, n, body1, (zero, zero))
    s2, b2 = _fold2(s2s, ss2s, m2, gb_ref[2:3, :cp], gb_ref[3:4, :cp])

    def body2(i, carry):
        s3s, ss3s, sss, ssss = carry
        z2 = jnp.maximum(
            y2_ref[i].astype(jnp.float32) * s2 + b2, 0.0).astype(jnp.bfloat16)
        y3 = jnp.dot(z2, w3_ref[...], preferred_element_type=jnp.float32
                     ).astype(jnp.bfloat16)
        y3_ref[i] = y3
        y3f = y3.astype(jnp.float32)
        ysc = jnp.dot(xs_ref[i, 0], ws_ref[...],
                      preferred_element_type=jnp.float32).astype(jnp.bfloat16)
        ysc_ref[i] = ysc
        yscf = ysc.astype(jnp.float32)
        return (s3s + jnp.sum(y3f, 0, keepdims=True),
                ss3s + jnp.sum(y3f * y3f, 0, keepdims=True),
                sss + jnp.sum(yscf, 0, keepdims=True),
                ssss + jnp.sum(yscf * yscf, 0, keepdims=True))

    zo = jnp.zeros((1, cpo), jnp.float32)
    s3s, ss3s, sss, ssss = jax.lax.fori_loop(
        0, n, body2, (zo, zo, zo, zo))
    s3, b3 = _fold2(s3s, ss3s, m2, gb_ref[4:5], gb_ref[5:6])
    ssc, bsc = _fold2(sss, ssss, m2, gb_ref[6:7], gb_ref[7:8])

    half = n // 2
    c = pl.program_id(0)
    for j in range(half):
        i = half * c + j
        a = y3_ref[i].astype(jnp.float32) * s3 + b3
        b = ysc_ref[i].astype(jnp.float32) * ssc + bsc
        o_ref[pl.ds(j * hw, hw), :] = jnp.maximum(a + b, 0.0)


def kernel(x, w1, g1, b1, w2, g2, b2, w3, g3, b3, ws, gs, bs):
    N, Cin, H, W = x.shape
    planes = w1.shape[0]
    cout = w3.shape[0]
    cp = _rup(planes, LANE)
    cpo = _rup(cout, LANE)
    Ho, Wo = (H - 1) // 2 + 1, (W - 1) // 2 + 1
    hw = Ho * Wo
    M1, M2 = N * H * W, N * hw
    bf = jnp.bfloat16
    f32 = jnp.float32

    w1m = _pad_last(w1[:, :, 0, 0].T, cp).astype(bf)
    w2t = jnp.transpose(w2, (2, 3, 1, 0))
    w2m = jnp.pad(
        w2t, ((0, 0), (0, 0), (0, cp - planes), (0, cp - planes))
    ).reshape(3, 3 * cp, cp).astype(bf)
    w3m = jnp.pad(
        w3[:, :, 0, 0].T, ((0, cp - planes), (0, cpo - cout))).astype(bf)
    wsm = _pad_last(ws[:, :, 0, 0].T, cpo).astype(bf)
    gb = jnp.stack([
        _pad_last(g1, cpo), _pad_last(b1, cpo),
        _pad_last(g2, cpo), _pad_last(b2, cpo),
        _pad_last(g3, cpo), _pad_last(b3, cpo),
        _pad_last(gs, cpo), _pad_last(bs, cpo)])

    x2d = jnp.transpose(
        x.reshape(N, Cin, Ho, 2, Wo, 2), (0, 3, 5, 2, 4, 1)
    ).astype(bf).reshape(M1, Cin)

    TM1 = _tile(M1, 4 * hw)
    gr1 = M1 // TM1
    y1, st1 = pl.pallas_call(
        _mm_stats_kernel,
        grid=(gr1,),
        in_specs=[pl.BlockSpec((TM1, Cin), lambda i: (i, 0)),
                  pl.BlockSpec((Cin, cp), lambda i: (0, 0))],
        out_specs=[pl.BlockSpec((TM1, cp), lambda i: (i, 0)),
                   pl.BlockSpec((1, 2, cp), lambda i: (i, 0, 0))],
        out_shape=[jax.ShapeDtypeStruct((M1, cp), bf),
                   jax.ShapeDtypeStruct((gr1, 2, cp), f32)],
        compiler_params=_CP,
    )(x2d, w1m)

    mk = functools.partial(
        _mega_kernel, n=N, ho=Ho, wo=Wo, m1=float(M1), m2=float(M2),
        cp=cp, cpo=cpo)
    out2d = pl.pallas_call(
        mk,
        grid=(2,),
        in_specs=[pl.BlockSpec((N, 4, hw, cp), lambda c: (0, 0, 0, 0)),
                  pl.BlockSpec((gr1, 2, cp), lambda c: (0, 0, 0)),
                  pl.BlockSpec((N, 1, hw, Cin), lambda c: (0, 0, 0, 0)),
                  pl.BlockSpec((3, 3 * cp, cp), lambda c: (0, 0, 0)),
                  pl.BlockSpec((cp, cpo), lambda c: (0, 0)),
                  pl.BlockSpec((Cin, cpo), lambda c: (0, 0)),
                  pl.BlockSpec((8, cpo), lambda c: (0, 0))],
        out_specs=pl.BlockSpec((M2 // 2, cpo), lambda c: (c, 0)),
        out_shape=jax.ShapeDtypeStruct((M2, cpo), f32),
        scratch_shapes=[pltpu.VMEM((4, hw, cp), bf),
                        pltpu.VMEM((N, hw, cp), bf),
                        pltpu.VMEM((N, hw, cpo), bf),
                        pltpu.VMEM((N, hw, cpo), bf)],
        compiler_params=_CP,
    )(y1.reshape(N, 4, hw, cp), st1, x2d.reshape(N, 4, hw, Cin),
      w2m, w3m, wsm, gb)

    out = out2d[:, :cout].reshape(N, Ho, Wo, cout)
    return jnp.transpose(out, (0, 3, 1, 2))


# speedup vs baseline: 1.1877x; 1.1877x over previous
import functools

import jax
import jax.numpy as jnp
from jax.experimental import pallas as pl
from jax.experimental.pallas import tpu as pltpu

EPS = 1e-5
LANE = 128

_CP = pltpu.CompilerParams(
    dimension_semantics=("parallel",),
    vmem_limit_bytes=64 * 1024 * 1024,
)


def _rup(x, m):
    return (x + m - 1) // m * m


def _tile(m, target):
    if m <= target:
        return m
    for t in range(target, 7, -1):
        if m % t == 0 and t % 8 == 0:
            return t
    return m


def _pad_last(a, c):
    pad = c - a.shape[-1]
    if pad == 0:
        return a
    return jnp.pad(a, [(0, 0)] * (a.ndim - 1) + [(0, pad)])


def _stats(yf):
    return jnp.concatenate(
        [jnp.sum(yf, 0, keepdims=True), jnp.sum(yf * yf, 0, keepdims=True)], 0)


def _fold2(s, ss, count, gamma, beta):
    mean = s / count
    var = jnp.maximum(ss / count - mean * mean, 0.0)
    scale = gamma * jax.lax.rsqrt(var + EPS)
    return scale, beta - mean * scale


def _fold(st, count, gamma, beta):
    s = jnp.sum(st[:, 0, :], axis=0, keepdims=True)
    ss = jnp.sum(st[:, 1, :], axis=0, keepdims=True)
    return _fold2(s, ss, count, gamma, beta)


def _csum(v):
    return (jnp.sum(v, axis=(0, 1)).reshape(1, -1),
            jnp.sum(v * v, axis=(0, 1)).reshape(1, -1))


def _dot2(a3, w):
    return jax.lax.dot_general(
        a3, w, dimension_numbers=(((2,), (0,)), ((), ())),
        preferred_element_type=jnp.float32)


def _mm_stats_kernel(x_ref, w_ref, y_ref, st_ref):
    y = jnp.dot(x_ref[...], w_ref[...], preferred_element_type=jnp.float32)
    y_ref[...] = y
    st_ref[0] = _stats(y)


def _mega_kernel(y1_ref, st1_ref, xs_ref, w2_ref, w3_ref, ws_ref, gb_ref,
                 o_ref, z_ref, y2_ref, y3_ref, ysc_ref,
                 *, n, ho, wo, m1, m2, cp, cpo):
    s1, b1 = _fold(st1_ref[...], m1,
                   gb_ref[0:1, :cp], gb_ref[1:2, :cp])
    s1 = s1.reshape(1, 1, cp)
    b1 = b1.reshape(1, 1, cp)

    def body1(i, carry):
        s2s, ss2s = carry
        for a in range(2):
            for b in range(2):
                yp = y1_ref[i, pl.Slice(a, ho, 2), pl.Slice(b, wo, 2), :]
                z_ref[a * 2 + b] = jnp.maximum(
                    yp * s1 + b1, 0.0).astype(jnp.bfloat16)
        acc = jnp.zeros((ho, wo, cp), jnp.float32)
        for ky in range(3):
            parts = []
            for kx in range(3):
                a, b = (ky + 1) % 2, (kx + 1) % 2
                base = z_ref[a * 2 + b]
                if ky == 0:
                    base = jnp.concatenate(
                        [jnp.zeros((1, wo, cp), base.dtype), base[:ho - 1]], 0)
                if kx == 0:
                    base = jnp.concatenate(
                        [jnp.zeros((ho, 1, cp), base.dtype),
                         base[:, :wo - 1]], 1)
                parts.append(base)
            wide = jnp.concatenate(parts, axis=2)
            acc = acc + _dot2(wide, w2_ref[ky])
        yb = acc.astype(jnp.bfloat16)
        y2_ref[i] = yb
        s, ss = _csum(yb.astype(jnp.float32))
        return (s2s + s, ss2s + ss)

    zero = jnp.zeros((1, cp), jnp.float32)
    s2s, ss2s = jax.lax.fori_loop(0, n, body1, (zero, zero))
    s2, b2 = _fold2(s2s, ss2s, m2, gb_ref[2:3, :cp], gb_ref[3:4, :cp])
    s2 = s2.reshape(1, 1, cp)
    b2 = b2.reshape(1, 1, cp)

    def body2(i, carry):
        s3s, ss3s, sss, ssss = carry
        z2 = jnp.maximum(
            y2_ref[i].astype(jnp.float32) * s2 + b2, 0.0).astype(jnp.bfloat16)
        y3 = _dot2(z2, w3_ref[...]).astype(jnp.bfloat16)
        y3_ref[i] = y3
        xs = xs_ref[i]
        ysc = _dot2(xs, ws_ref[...]).astype(jnp.bfloat16)
        ysc_ref[i] = ysc
        s3, ss3 = _csum(y3.astype(jnp.float32))
        s4, ss4 = _csum(ysc.astype(jnp.float32))
        return (s3s + s3, ss3s + ss3, sss + s4, ssss + ss4)

    zo = jnp.zeros((1, cpo), jnp.float32)
    s3s, ss3s, sss, ssss = jax.lax.fori_loop(0, n, body2, (zo, zo, zo, zo))
    s3, b3 = _fold2(s3s, ss3s, m2, gb_ref[4:5], gb_ref[5:6])
    ssc, bsc = _fold2(sss, ssss, m2, gb_ref[6:7], gb_ref[7:8])
    s3 = s3.reshape(1, 1, cpo)
    b3 = b3.reshape(1, 1, cpo)
    ssc = ssc.reshape(1, 1, cpo)
    bsc = bsc.reshape(1, 1, cpo)

    half = n // 2
    c = pl.program_id(0)
    for j in range(half):
        i = half * c + j
        a = y3_ref[i].astype(jnp.float32) * s3 + b3
        b = ysc_ref[i].astype(jnp.float32) * ssc + bsc
        o_ref[j] = jnp.maximum(a + b, 0.0)


def kernel(x, w1, g1, b1, w2, g2, b2, w3, g3, b3, ws, gs, bs):
    N, Cin, H, W = x.shape
    planes = w1.shape[0]
    cout = w3.shape[0]
    cp = _rup(planes, LANE)
    cpo = _rup(cout, LANE)
    Ho, Wo = (H - 1) // 2 + 1, (W - 1) // 2 + 1
    hw = Ho * Wo
    M1, M2 = N * H * W, N * hw
    bf = jnp.bfloat16
    f32 = jnp.float32

    w1m = _pad_last(w1[:, :, 0, 0].T, cp).astype(bf)
    w2t = jnp.transpose(w2, (2, 3, 1, 0))
    w2m = jnp.pad(
        w2t, ((0, 0), (0, 0), (0, cp - planes), (0, cp - planes))
    ).reshape(3, 3 * cp, cp).astype(bf)
    w3m = jnp.pad(
        w3[:, :, 0, 0].T, ((0, cp - planes), (0, cpo - cout))).astype(bf)
    wsm = _pad_last(ws[:, :, 0, 0].T, cpo).astype(bf)
    gb = jnp.stack([
        _pad_last(g1, cpo), _pad_last(b1, cpo),
        _pad_last(g2, cpo), _pad_last(b2, cpo),
        _pad_last(g3, cpo), _pad_last(b3, cpo),
        _pad_last(gs, cpo), _pad_last(bs, cpo)])

    xt = jnp.transpose(x, (0, 2, 3, 1)).astype(bf)
    x2d = xt.reshape(M1, Cin)

    TM1 = _tile(M1, 4 * hw)
    gr1 = M1 // TM1
    y1, st1 = pl.pallas_call(
        _mm_stats_kernel,
        grid=(gr1,),
        in_specs=[pl.BlockSpec((TM1, Cin), lambda i: (i, 0)),
                  pl.BlockSpec((Cin, cp), lambda i: (0, 0))],
        out_specs=[pl.BlockSpec((TM1, cp), lambda i: (i, 0)),
                   pl.BlockSpec((1, 2, cp), lambda i: (i, 0, 0))],
        out_shape=[jax.ShapeDtypeStruct((M1, cp), f32),
                   jax.ShapeDtypeStruct((gr1, 2, cp), f32)],
        compiler_params=_CP,
    )(x2d, w1m)

    xs4 = xt.reshape(N, Ho, 2, Wo, 2, Cin)[:, :, 0, :, 0, :]
    mk = functools.partial(
        _mega_kernel, n=N, ho=Ho, wo=Wo, m1=float(M1), m2=float(M2),
        cp=cp, cpo=cpo)
    out = pl.pallas_call(
        mk,
        grid=(2,),
        in_specs=[pl.BlockSpec((N, H, W, cp), lambda c: (0, 0, 0, 0)),
                  pl.BlockSpec((gr1, 2, cp), lambda c: (0, 0, 0)),
                  pl.BlockSpec((N, Ho, Wo, Cin), lambda c: (0, 0, 0, 0)),
                  pl.BlockSpec((3, 3 * cp, cp), lambda c: (0, 0, 0)),
                  pl.BlockSpec((cp, cpo), lambda c: (0, 0)),
                  pl.BlockSpec((Cin, cpo), lambda c: (0, 0)),
                  pl.BlockSpec((8, cpo), lambda c: (0, 0))],
        out_specs=pl.BlockSpec((N // 2, Ho, Wo, cpo), lambda c: (c, 0, 0, 0)),
        out_shape=jax.ShapeDtypeStruct((N, Ho, Wo, cpo), f32),
        scratch_shapes=[pltpu.VMEM((4, Ho, Wo, cp), bf),
                        pltpu.VMEM((N, Ho, Wo, cp), bf),
                        pltpu.VMEM((N, Ho, Wo, cpo), bf),
                        pltpu.VMEM((N, Ho, Wo, cpo), bf)],
        compiler_params=_CP,
    )(y1.reshape(N, H, W, cp), st1, xs4, w2m, w3m, wsm, gb)

    return jnp.transpose(out[..., :cout], (0, 3, 1, 2))
